# baseline (speedup 1.0000x reference)
import jax
import jax.numpy as jnp
from jax.experimental import pallas as pl
from jax.experimental.pallas import tpu as pltpu


def _copy_block(x_ref, o_ref):
    o_ref[...] = x_ref[...]


def kernel(x):
    shape = x.shape
    total = x.size
    lanes = 1024
    rows = total // lanes
    xf = x.reshape(rows, lanes)
    grid = 15
    block = (rows // grid, lanes)
    out = pl.pallas_call(
        _copy_block,
        grid=(grid,),
        in_specs=[pl.BlockSpec(block, lambda i: (i, 0))],
        out_specs=pl.BlockSpec(block, lambda i: (i, 0)),
        out_shape=jax.ShapeDtypeStruct((rows, lanes), x.dtype),
    )(xf)
    return out.reshape(shape)


# speedup vs baseline: 1.3301x; 1.3301x over previous
import jax
import jax.numpy as jnp
from jax.experimental import pallas as pl
from jax.experimental.pallas import tpu as pltpu

_B, _N, _C, _H, _D = 8, 1025, 3, 16, 64
_NJ = 5
_CH = _N // _NJ
_NCHUNK = _B * _NJ
_K = 8


def _copy_body(x_ref, o_ref, buf, in_sems, out_sems):
    def src(i):
        return x_ref.at[i // _NJ, pl.ds((i % _NJ) * _CH, _CH)]

    def dst(i):
        return o_ref.at[i // _NJ, pl.ds((i % _NJ) * _CH, _CH)]

    for k in range(_K):
        pltpu.make_async_copy(src(k), buf.at[k], in_sems.at[k]).start()

    for g in range(_NCHUNK // _K):
        base = g * _K
        for k in range(_K):
            i = base + k
            pltpu.make_async_copy(src(i), buf.at[k], in_sems.at[k]).wait()
            pltpu.make_async_copy(buf.at[k], dst(i), out_sems.at[k]).start()
        for k in range(_K):
            i = base + k
            pltpu.make_async_copy(buf.at[k], dst(i), out_sems.at[k]).wait()
            if i + _K < _NCHUNK:
                pltpu.make_async_copy(
                    src(i + _K), buf.at[k], in_sems.at[k]
                ).start()


def kernel(x):
    return pl.pallas_call(
        _copy_body,
        in_specs=[pl.BlockSpec(memory_space=pl.ANY)],
        out_specs=pl.BlockSpec(memory_space=pl.ANY),
        out_shape=jax.ShapeDtypeStruct(x.shape, x.dtype),
        scratch_shapes=[
            pltpu.VMEM((_K, _CH, _C, _H, _D), jnp.float32),
            pltpu.SemaphoreType.DMA((_K,)),
            pltpu.SemaphoreType.DMA((_K,)),
        ],
    )(x)
